# baseline (speedup 1.0000x reference)
import functools

import jax
import jax.numpy as jnp
from jax import lax
from jax.experimental import pallas as pl
from jax.experimental.pallas import tpu as pltpu
from jax.experimental.pallas import tpu_sc as plsc

B = 16384
D = 64
H = 16

NC = 2
NS = 16
NW = NC * NS
BPW = B // NW
CH = 64
NCH = BPW // CH


def _sc_gather(tablep, idx1):
    mesh = plsc.VectorSubcoreMesh(core_axis_name="c", subcore_axis_name="s")

    @functools.partial(
        pl.kernel,
        out_type=jax.ShapeDtypeStruct((B, D), jnp.float32),
        mesh=mesh,
        scratch_types=[
            pltpu.VMEM((BPW,), jnp.int32),
            pltpu.VMEM((NCH, CH), jnp.int32),
            pltpu.VMEM((BPW,), jnp.int32),
            pltpu.VMEM((2, CH, 2 * D), jnp.float32),
            pltpu.VMEM((BPW, D), jnp.float32),
            [pltpu.SemaphoreType.DMA] * 2,
        ],
        compiler_params=pltpu.CompilerParams(needs_layout_passes=False),
    )
    def k(tab_hbm, idx_hbm, out_hbm, idx_v, q_v, r_v, pair_v, rows_v, sems):
        wid = lax.axis_index("s") * NC + lax.axis_index("c")
        base = wid * BPW
        pltpu.sync_copy(idx_hbm.at[pl.ds(base, BPW)], idx_v)

        def qb(g, carry):
            v = idx_v[pl.ds(g * 16, 16)]
            r_v[pl.ds(g * 16, 16)] = v & 1
            return carry

        lax.fori_loop(0, BPW // 16, qb, 0)
        for ch in range(NCH):
            def qb2(g, carry, ch=ch):
                v = idx_v[pl.ds(ch * CH + g * 16, 16)]
                q_v[ch, pl.ds(g * 16, 16)] = v >> 1
                return carry
            lax.fori_loop(0, CH // 16, qb2, 0)

        def issue(ch):
            pltpu.async_copy(
                tab_hbm.at[q_v.at[ch]], pair_v.at[ch % 2], sems[ch % 2]
            )

        issue(0)
        for ch in range(NCH):
            buf = ch % 2
            pltpu.make_async_copy(
                tab_hbm.at[q_v.at[ch]], pair_v.at[buf], sems[buf]
            ).wait()
            if ch + 1 < NCH:
                issue(ch + 1)

            def ebody(t, carry, ch=ch, buf=buf):
                sel = plsc.load_gather(
                    r_v, [jnp.full((16,), ch * CH, jnp.int32) + t]
                )
                pick_lo = sel == 0
                for kk in range(D // 16):
                    lo = pair_v[buf, t, pl.ds(kk * 16, 16)]
                    hi = pair_v[buf, t, pl.ds(D + kk * 16, 16)]
                    rows_v[ch * CH + t, pl.ds(kk * 16, 16)] = jnp.where(
                        pick_lo, lo, hi
                    )
                return carry

            lax.fori_loop(0, CH, ebody, 0)

        pltpu.sync_copy(rows_v, out_hbm.at[pl.ds(base, BPW)])

    return k(tablep, idx1)


def _mlp_body(x_ref, w1_ref, b1_ref, w2_ref, b2_ref, oe_ref, os_ref):
    x = x_ref[...]
    h = jnp.dot(x, w1_ref[...], preferred_element_type=jnp.float32)
    h = jnp.maximum(h + b1_ref[...], 0.0)
    y = (
        jnp.dot(h, w2_ref[...], preferred_element_type=jnp.float32)
        + b2_ref[...]
    )
    oe_ref[...] = x.T
    os_ref[...] = y.T


def _tc_mlp(embs, W1, b1, W2, b2):
    bm = 2048
    return pl.pallas_call(
        _mlp_body,
        grid=(B // bm,),
        in_specs=[
            pl.BlockSpec((bm, D), lambda i: (i, 0)),
            pl.BlockSpec((D, H), lambda i: (0, 0)),
            pl.BlockSpec((1, H), lambda i: (0, 0)),
            pl.BlockSpec((H, D), lambda i: (0, 0)),
            pl.BlockSpec((1, D), lambda i: (0, 0)),
        ],
        out_specs=[
            pl.BlockSpec((D, bm), lambda i: (0, i)),
            pl.BlockSpec((D, bm), lambda i: (0, i)),
        ],
        out_shape=[
            jax.ShapeDtypeStruct((D, B), jnp.float32),
            jax.ShapeDtypeStruct((D, B), jnp.float32),
        ],
    )(embs, W1, b1.reshape(1, H), W2, b2.reshape(1, D))


def kernel(states, table, W1, b1, W2, b2):
    idx1 = states.reshape(B).astype(jnp.int32)
    tablep = table.reshape(500000, 2 * D)
    embs = _sc_gather(tablep, idx1)
    embsT, sfsT = _tc_mlp(embs, W1, b1, W2, b2)
    return (embsT.T, sfsT.T)


# speedup vs baseline: 2.0075x; 2.0075x over previous
import functools

import jax
import jax.numpy as jnp
from jax import lax
from jax.experimental import pallas as pl
from jax.experimental.pallas import tpu as pltpu
from jax.experimental.pallas import tpu_sc as plsc

B = 16384
D = 64
H = 16
V = 1000000

NC = 2
NS = 16
NW = NC * NS
BPW = B // NW
CH = 64
NCH = BPW // CH

NBLK = (V + 127) // 128
BPB = (NBLK + NW - 1) // NW
LAST = NBLK - 1
SLAB = 1024


def _sget(ref, lanes, i):
    v = ref[pl.ds((i >> 4) << 4, 16)]
    return jnp.max(jnp.where(lanes == (i & 15), v, jnp.int32(-1)))


def _sc_sweep(tableT, tailT, sidx, bounds):
    mesh = plsc.VectorSubcoreMesh(core_axis_name="c", subcore_axis_name="s")

    @functools.partial(
        pl.kernel,
        out_type=jax.ShapeDtypeStruct((B * D,), jnp.float32),
        mesh=mesh,
        scratch_types=[
            pltpu.VMEM((B + 16,), jnp.int32),
            pltpu.VMEM((48,), jnp.int32),
            pltpu.VMEM((2, D, 128), jnp.float32),
            pltpu.VMEM((SLAB // 2, 128), jnp.float32),
            [pltpu.SemaphoreType.DMA] * 2,
            pltpu.SemaphoreType.DMA,
        ],
        compiler_params=pltpu.CompilerParams(needs_layout_passes=False),
    )
    def k(tab_hbm, tail_hbm, sidx_hbm, bnd_hbm, out_hbm, sidx_v, bnd_v,
          blk_v, slab_v, bsems, osem):
        wid = lax.axis_index("s") * NC + lax.axis_index("c")
        lanes = lax.iota(jnp.int32, 16)
        pltpu.sync_copy(sidx_hbm, sidx_v.at[pl.ds(0, B)])
        pltpu.sync_copy(bnd_hbm, bnd_v)
        lo = _sget(bnd_v, lanes, wid)
        hi = _sget(bnd_v, lanes, wid + 1)
        b0 = wid * BPB
        b1 = jnp.minimum(b0 + BPB, NBLK)

        def issue(cb, buf):
            @pl.when(cb < LAST)
            def _():
                pltpu.async_copy(
                    tab_hbm.at[:, pl.ds(cb * 128, 128)],
                    blk_v.at[buf], bsems[buf],
                )

            @pl.when(cb == LAST)
            def _():
                pltpu.async_copy(tail_hbm, blk_v.at[buf], bsems[buf])

        def wait(cb, buf):
            pltpu.make_async_copy(
                tab_hbm.at[:, pl.ds(0, 128)], blk_v.at[buf], bsems[buf]
            ).wait()

        @pl.when(b0 < b1)
        def _():
            issue(b0, 0)

        def one_block(cb, buf, p):
            def live():
                wait(cb, buf)

                @pl.when(cb + 1 < b1)
                def _():
                    issue(cb + 1, 1 - buf)

                end = (cb + 1) * 128

                def cond(p2):
                    return jnp.logical_and(
                        p2 < hi, _sget(sidx_v, lanes, p2) < end
                    )

                def hit(p2):
                    c = _sget(sidx_v, lanes, p2) - cb * 128
                    ph = p2 - lo
                    for kk in range(D // 16):
                        col = plsc.load_gather(
                            blk_v,
                            [
                                jnp.full((16,), buf, jnp.int32),
                                lanes + kk * 16,
                                jnp.full((16,), 0, jnp.int32) + c,
                            ],
                        )
                        slab_v[ph >> 1, pl.ds((ph & 1) * D + kk * 16, 16)] = (
                            col
                        )
                    return p2 + 1

                return lax.while_loop(cond, hit, p)

            return lax.cond(cb < b1, live, lambda: p)

        def pair_body(u, p):
            cb = b0 + 2 * u
            p = one_block(cb, 0, p)
            p = one_block(cb + 1, 1, p)
            return p

        p_end = lax.fori_loop(0, (BPB + 1) // 2, pair_body, lo)
        n = p_end - lo

        def obody(t, carry):
            @pl.when(t < n)
            def _():
                pltpu.async_copy(
                    slab_v.at[t >> 1, pl.ds((t & 1) * D, D)],
                    out_hbm.at[pl.ds((lo + t) * D, D)],
                    osem,
                )
            return carry

        lax.fori_loop(0, SLAB, obody, 0)

        def dbody(t, carry):
            @pl.when(t < n)
            def _():
                pltpu.make_async_copy(
                    out_hbm.at[pl.ds(0, D)],
                    slab_v.at[0, pl.ds(0, D)],
                    osem,
                ).wait()
            return carry

        lax.fori_loop(0, SLAB, dbody, 0)

    return k(tableT, tailT, sidx, bounds)


def _sc_unsort(spair, rank):
    mesh = plsc.VectorSubcoreMesh(core_axis_name="c", subcore_axis_name="s")

    @functools.partial(
        pl.kernel,
        out_type=jax.ShapeDtypeStruct((B, D), jnp.float32),
        mesh=mesh,
        scratch_types=[
            pltpu.VMEM((BPW,), jnp.int32),
            pltpu.VMEM((NCH, CH), jnp.int32),
            pltpu.VMEM((BPW,), jnp.int32),
            pltpu.VMEM((2, CH, 2 * D), jnp.float32),
            pltpu.VMEM((BPW, D), jnp.float32),
            [pltpu.SemaphoreType.DMA] * 2,
        ],
        compiler_params=pltpu.CompilerParams(needs_layout_passes=False),
    )
    def k(tab_hbm, idx_hbm, out_hbm, idx_v, q_v, r_v, pair_v, rows_v, sems):
        wid = lax.axis_index("s") * NC + lax.axis_index("c")
        base = wid * BPW
        pltpu.sync_copy(idx_hbm.at[pl.ds(base, BPW)], idx_v)

        def qb(g, carry):
            v = idx_v[pl.ds(g * 16, 16)]
            r_v[pl.ds(g * 16, 16)] = v & 1
            return carry

        lax.fori_loop(0, BPW // 16, qb, 0)
        for ch in range(NCH):
            def qb2(g, carry, ch=ch):
                v = idx_v[pl.ds(ch * CH + g * 16, 16)]
                q_v[ch, pl.ds(g * 16, 16)] = v >> 1
                return carry
            lax.fori_loop(0, CH // 16, qb2, 0)

        def issue(ch):
            pltpu.async_copy(
                tab_hbm.at[q_v.at[ch]], pair_v.at[ch % 2], sems[ch % 2]
            )

        issue(0)
        for ch in range(NCH):
            buf = ch % 2
            pltpu.make_async_copy(
                tab_hbm.at[q_v.at[ch]], pair_v.at[buf], sems[buf]
            ).wait()
            if ch + 1 < NCH:
                issue(ch + 1)

            def ebody(t, carry, ch=ch, buf=buf):
                sel = plsc.load_gather(
                    r_v, [jnp.full((16,), ch * CH, jnp.int32) + t]
                )
                pick_lo = sel == 0
                for kk in range(D // 16):
                    lo = pair_v[buf, t, pl.ds(kk * 16, 16)]
                    hi = pair_v[buf, t, pl.ds(D + kk * 16, 16)]
                    rows_v[ch * CH + t, pl.ds(kk * 16, 16)] = jnp.where(
                        pick_lo, lo, hi
                    )
                return carry

            lax.fori_loop(0, CH, ebody, 0)

        pltpu.sync_copy(rows_v, out_hbm.at[pl.ds(base, BPW)])

    return k(spair, rank)


def _mlp_body(x_ref, w1_ref, b1_ref, w2_ref, b2_ref, oe_ref, os_ref):
    x = x_ref[...]
    h = jnp.dot(x, w1_ref[...], preferred_element_type=jnp.float32)
    h = jnp.maximum(h + b1_ref[...], 0.0)
    y = (
        jnp.dot(h, w2_ref[...], preferred_element_type=jnp.float32)
        + b2_ref[...]
    )
    oe_ref[...] = x.T
    os_ref[...] = y.T


def _tc_mlp(embs, W1, b1, W2, b2):
    bm = 2048
    return pl.pallas_call(
        _mlp_body,
        grid=(B // bm,),
        in_specs=[
            pl.BlockSpec((bm, D), lambda i: (i, 0)),
            pl.BlockSpec((D, H), lambda i: (0, 0)),
            pl.BlockSpec((1, H), lambda i: (0, 0)),
            pl.BlockSpec((H, D), lambda i: (0, 0)),
            pl.BlockSpec((1, D), lambda i: (0, 0)),
        ],
        out_specs=[
            pl.BlockSpec((D, bm), lambda i: (0, i)),
            pl.BlockSpec((D, bm), lambda i: (0, i)),
        ],
        out_shape=[
            jax.ShapeDtypeStruct((D, B), jnp.float32),
            jax.ShapeDtypeStruct((D, B), jnp.float32),
        ],
    )(embs, W1, b1.reshape(1, H), W2, b2.reshape(1, D))


def kernel(states, table, W1, b1, W2, b2):
    idx1 = states.reshape(B).astype(jnp.int32)
    iota = lax.iota(jnp.int32, B)
    sidx, perm = lax.sort([idx1, iota], num_keys=1)
    rank = jnp.zeros((B,), jnp.int32).at[perm].set(iota)
    starts = (lax.iota(jnp.int32, 48) * (BPB * 128)).astype(jnp.int32)
    bounds = jnp.searchsorted(sidx, starts).astype(jnp.int32)
    tableT = table.T
    tailT = jnp.pad(tableT[:, V - 64:], ((0, 0), (0, 64)))
    sorted_flat = _sc_sweep(tableT, tailT, sidx, bounds)
    spair = sorted_flat.reshape(B * D // 128, 128)
    embs = _sc_unsort(spair, rank)
    embsT, sfsT = _tc_mlp(embs, W1, b1, W2, b2)
    return (embsT.T, sfsT.T)


# speedup vs baseline: 3.8904x; 1.9380x over previous
import functools

import jax
import jax.numpy as jnp
from jax import lax
from jax.experimental import pallas as pl
from jax.experimental.pallas import tpu as pltpu
from jax.experimental.pallas import tpu_sc as plsc

B = 16384
D = 64
H = 16
V = 1000000

NC = 2
NS = 16
NW = NC * NS
BPW = B // NW
CH = 64
NCH = BPW // CH

NBLK = (V + 127) // 128
BPB = (NBLK + NW - 1) // NW
LAST = NBLK - 1
SLAB = 1008
NBUF = 6


def _sget(ref, lanes, i):
    v = ref[pl.ds((i >> 4) << 4, 16)]
    return jnp.max(jnp.where(lanes == (i & 15), v, jnp.int32(-1)))


def _sc_sweep(tableT, tailT, sidx, bounds):
    mesh = plsc.VectorSubcoreMesh(core_axis_name="c", subcore_axis_name="s")

    @functools.partial(
        pl.kernel,
        out_type=jax.ShapeDtypeStruct((B * D,), jnp.float32),
        mesh=mesh,
        scratch_types=[
            pltpu.VMEM((B + 16,), jnp.int32),
            pltpu.VMEM((48,), jnp.int32),
            pltpu.VMEM((NBUF, D, 128), jnp.float32),
            pltpu.VMEM((SLAB // 2, 128), jnp.float32),
            [pltpu.SemaphoreType.DMA] * NBUF,
            pltpu.SemaphoreType.DMA,
        ],
        compiler_params=pltpu.CompilerParams(needs_layout_passes=False),
    )
    def k(tab_hbm, tail_hbm, sidx_hbm, bnd_hbm, out_hbm, sidx_v, bnd_v,
          blk_v, slab_v, bsems, osem):
        wid = lax.axis_index("s") * NC + lax.axis_index("c")
        lanes = lax.iota(jnp.int32, 16)
        pltpu.sync_copy(sidx_hbm, sidx_v.at[pl.ds(0, B)])
        pltpu.sync_copy(bnd_hbm, bnd_v)
        lo = _sget(bnd_v, lanes, wid)
        hi = _sget(bnd_v, lanes, wid + 1)
        b0 = wid * BPB
        b1 = jnp.minimum(b0 + BPB, NBLK)

        def issue(cb, buf):
            @pl.when(cb < LAST)
            def _():
                pltpu.async_copy(
                    tab_hbm.at[:, pl.ds(cb * 128, 128)],
                    blk_v.at[buf], bsems[buf],
                )

            @pl.when(cb == LAST)
            def _():
                pltpu.async_copy(tail_hbm, blk_v.at[buf], bsems[buf])

        def wait(cb, buf):
            pltpu.make_async_copy(
                tab_hbm.at[:, pl.ds(0, 128)], blk_v.at[buf], bsems[buf]
            ).wait()

        for s in range(NBUF - 1):
            @pl.when(b0 + s < b1)
            def _(s=s):
                issue(b0 + s, s)

        def one_block(cb, buf, p):
            def live():
                wait(cb, buf)

                @pl.when(cb + NBUF - 1 < b1)
                def _():
                    issue(cb + NBUF - 1, (buf + NBUF - 1) % NBUF)

                end = (cb + 1) * 128

                def cond(p2):
                    return jnp.logical_and(
                        p2 < hi, _sget(sidx_v, lanes, p2) < end
                    )

                def hit(p2):
                    c = _sget(sidx_v, lanes, p2) - cb * 128
                    ph = p2 - lo
                    for kk in range(D // 16):
                        col = plsc.load_gather(
                            blk_v,
                            [
                                jnp.full((16,), buf, jnp.int32),
                                lanes + kk * 16,
                                jnp.full((16,), 0, jnp.int32) + c,
                            ],
                        )
                        slab_v[ph >> 1, pl.ds((ph & 1) * D + kk * 16, 16)] = (
                            col
                        )
                    return p2 + 1

                return lax.while_loop(cond, hit, p)

            return lax.cond(cb < b1, live, lambda: p)

        def ring_body(u, p):
            for v in range(NBUF):
                p = one_block(b0 + NBUF * u + v, v, p)
            return p

        p_end = lax.fori_loop(0, (BPB + NBUF - 1) // NBUF, ring_body, lo)
        n = p_end - lo

        def obody(t, carry):
            @pl.when(t < n)
            def _():
                pltpu.async_copy(
                    slab_v.at[t >> 1, pl.ds((t & 1) * D, D)],
                    out_hbm.at[pl.ds((lo + t) * D, D)],
                    osem,
                )
            return carry

        lax.fori_loop(0, SLAB, obody, 0)

        def dbody(t, carry):
            @pl.when(t < n)
            def _():
                pltpu.make_async_copy(
                    out_hbm.at[pl.ds(0, D)],
                    slab_v.at[0, pl.ds(0, D)],
                    osem,
                ).wait()
            return carry

        lax.fori_loop(0, SLAB, dbody, 0)

    return k(tableT, tailT, sidx, bounds)


def _sc_unsort(spair, rank):
    mesh = plsc.VectorSubcoreMesh(core_axis_name="c", subcore_axis_name="s")

    @functools.partial(
        pl.kernel,
        out_type=jax.ShapeDtypeStruct((B, D), jnp.float32),
        mesh=mesh,
        scratch_types=[
            pltpu.VMEM((BPW,), jnp.int32),
            pltpu.VMEM((NCH, CH), jnp.int32),
            pltpu.VMEM((BPW,), jnp.int32),
            pltpu.VMEM((2, CH, 2 * D), jnp.float32),
            pltpu.VMEM((BPW, D), jnp.float32),
            [pltpu.SemaphoreType.DMA] * 2,
        ],
        compiler_params=pltpu.CompilerParams(needs_layout_passes=False),
    )
    def k(tab_hbm, idx_hbm, out_hbm, idx_v, q_v, r_v, pair_v, rows_v, sems):
        wid = lax.axis_index("s") * NC + lax.axis_index("c")
        base = wid * BPW
        pltpu.sync_copy(idx_hbm.at[pl.ds(base, BPW)], idx_v)

        def qb(g, carry):
            v = idx_v[pl.ds(g * 16, 16)]
            r_v[pl.ds(g * 16, 16)] = v & 1
            return carry

        lax.fori_loop(0, BPW // 16, qb, 0)
        for ch in range(NCH):
            def qb2(g, carry, ch=ch):
                v = idx_v[pl.ds(ch * CH + g * 16, 16)]
                q_v[ch, pl.ds(g * 16, 16)] = v >> 1
                return carry
            lax.fori_loop(0, CH // 16, qb2, 0)

        def issue(ch):
            pltpu.async_copy(
                tab_hbm.at[q_v.at[ch]], pair_v.at[ch % 2], sems[ch % 2]
            )

        issue(0)
        for ch in range(NCH):
            buf = ch % 2
            pltpu.make_async_copy(
                tab_hbm.at[q_v.at[ch]], pair_v.at[buf], sems[buf]
            ).wait()
            if ch + 1 < NCH:
                issue(ch + 1)

            def ebody(t, carry, ch=ch, buf=buf):
                sel = plsc.load_gather(
                    r_v, [jnp.full((16,), ch * CH, jnp.int32) + t]
                )
                pick_lo = sel == 0
                for kk in range(D // 16):
                    lo = pair_v[buf, t, pl.ds(kk * 16, 16)]
                    hi = pair_v[buf, t, pl.ds(D + kk * 16, 16)]
                    rows_v[ch * CH + t, pl.ds(kk * 16, 16)] = jnp.where(
                        pick_lo, lo, hi
                    )
                return carry

            lax.fori_loop(0, CH, ebody, 0)

        pltpu.sync_copy(rows_v, out_hbm.at[pl.ds(base, BPW)])

    return k(spair, rank)


def _mlp_body(x_ref, w1_ref, b1_ref, w2_ref, b2_ref, oe_ref, os_ref):
    x = x_ref[...]
    h = jnp.dot(x, w1_ref[...], preferred_element_type=jnp.float32)
    h = jnp.maximum(h + b1_ref[...], 0.0)
    y = (
        jnp.dot(h, w2_ref[...], preferred_element_type=jnp.float32)
        + b2_ref[...]
    )
    oe_ref[...] = x.T
    os_ref[...] = y.T


def _tc_mlp(embs, W1, b1, W2, b2):
    bm = 2048
    return pl.pallas_call(
        _mlp_body,
        grid=(B // bm,),
        in_specs=[
            pl.BlockSpec((bm, D), lambda i: (i, 0)),
            pl.BlockSpec((D, H), lambda i: (0, 0)),
            pl.BlockSpec((1, H), lambda i: (0, 0)),
            pl.BlockSpec((H, D), lambda i: (0, 0)),
            pl.BlockSpec((1, D), lambda i: (0, 0)),
        ],
        out_specs=[
            pl.BlockSpec((D, bm), lambda i: (0, i)),
            pl.BlockSpec((D, bm), lambda i: (0, i)),
        ],
        out_shape=[
            jax.ShapeDtypeStruct((D, B), jnp.float32),
            jax.ShapeDtypeStruct((D, B), jnp.float32),
        ],
    )(embs, W1, b1.reshape(1, H), W2, b2.reshape(1, D))


def kernel(states, table, W1, b1, W2, b2):
    idx1 = states.reshape(B).astype(jnp.int32)
    iota = lax.iota(jnp.int32, B)
    sidx, perm = lax.sort([idx1, iota], num_keys=1)
    rank = jnp.zeros((B,), jnp.int32).at[perm].set(iota)
    starts = (lax.iota(jnp.int32, 48) * (BPB * 128)).astype(jnp.int32)
    bounds = jnp.searchsorted(sidx, starts).astype(jnp.int32)
    tableT = table.T
    tailT = jnp.pad(tableT[:, V - 64:], ((0, 0), (0, 64)))
    sorted_flat = _sc_sweep(tableT, tailT, sidx, bounds)
    spair = sorted_flat.reshape(B * D // 128, 128)
    embs = _sc_unsort(spair, rank)
    embsT, sfsT = _tc_mlp(embs, W1, b1, W2, b2)
    return (embsT.T, sfsT.T)


# speedup vs baseline: 4.0180x; 1.0328x over previous
import functools

import jax
import jax.numpy as jnp
from jax import lax
from jax.experimental import pallas as pl
from jax.experimental.pallas import tpu as pltpu
from jax.experimental.pallas import tpu_sc as plsc

B = 16384
D = 64
H = 16
V = 1000000

NC = 2
NS = 16
NW = NC * NS
BPW = B // NW
CH = 64
NCH = BPW // CH

NBLK = (V + 127) // 128
BPB = (NBLK + NW - 1) // NW
LAST = NBLK - 1
SLAB = 992
NBUF = 8
SWIN = SLAB + 40


def _sget(ref, lanes, i):
    v = ref[pl.ds((i >> 4) << 4, 16)]
    return jnp.max(jnp.where(lanes == (i & 15), v, jnp.int32(-1)))


def _sc_sweep(tableT, tailT, sidx, bounds):
    mesh = plsc.VectorSubcoreMesh(core_axis_name="c", subcore_axis_name="s")

    @functools.partial(
        pl.kernel,
        out_type=jax.ShapeDtypeStruct((B * D,), jnp.float32),
        mesh=mesh,
        scratch_types=[
            pltpu.VMEM((SWIN,), jnp.int32),
            pltpu.VMEM((48,), jnp.int32),
            pltpu.VMEM((NBUF, D, 128), jnp.float32),
            pltpu.VMEM((SLAB // 2, 128), jnp.float32),
            [pltpu.SemaphoreType.DMA] * NBUF,
            pltpu.SemaphoreType.DMA,
        ],
        compiler_params=pltpu.CompilerParams(needs_layout_passes=False),
    )
    def k(tab_hbm, tail_hbm, sidx_hbm, bnd_hbm, out_hbm, sidx_v, bnd_v,
          blk_v, slab_v, bsems, osem):
        wid = lax.axis_index("s") * NC + lax.axis_index("c")
        lanes = lax.iota(jnp.int32, 16)
        pltpu.sync_copy(bnd_hbm, bnd_v)
        lo = _sget(bnd_v, lanes, wid)
        hi = _sget(bnd_v, lanes, wid + 1)
        lo16 = pl.multiple_of((lo >> 4) << 4, 16)
        pltpu.sync_copy(sidx_hbm.at[pl.ds(lo16, SWIN)], sidx_v)
        b0 = wid * BPB
        b1 = jnp.minimum(b0 + BPB, NBLK)

        def issue(cb, buf):
            @pl.when(cb < LAST)
            def _():
                pltpu.async_copy(
                    tab_hbm.at[:, pl.ds(cb * 128, 128)],
                    blk_v.at[buf], bsems[buf],
                )

            @pl.when(cb == LAST)
            def _():
                pltpu.async_copy(tail_hbm, blk_v.at[buf], bsems[buf])

        def wait(cb, buf):
            pltpu.make_async_copy(
                tab_hbm.at[:, pl.ds(0, 128)], blk_v.at[buf], bsems[buf]
            ).wait()

        for s in range(NBUF - 1):
            @pl.when(b0 + s < b1)
            def _(s=s):
                issue(b0 + s, s)

        def one_block(cb, buf, p):
            def live():
                wait(cb, buf)

                @pl.when(cb + NBUF - 1 < b1)
                def _():
                    issue(cb + NBUF - 1, (buf + NBUF - 1) % NBUF)

                end = (cb + 1) * 128

                def cond(p2):
                    return jnp.logical_and(
                        p2 < hi, _sget(sidx_v, lanes, p2 - lo16) < end
                    )

                def hit(p2):
                    c = _sget(sidx_v, lanes, p2 - lo16) - cb * 128
                    ph = p2 - lo
                    for kk in range(D // 16):
                        col = plsc.load_gather(
                            blk_v,
                            [
                                jnp.full((16,), buf, jnp.int32),
                                lanes + kk * 16,
                                jnp.full((16,), 0, jnp.int32) + c,
                            ],
                        )
                        slab_v[ph >> 1, pl.ds((ph & 1) * D + kk * 16, 16)] = (
                            col
                        )
                    return p2 + 1

                return lax.while_loop(cond, hit, p)

            return lax.cond(cb < b1, live, lambda: p)

        def ring_body(u, p):
            for v in range(NBUF):
                p = one_block(b0 + NBUF * u + v, v, p)
            return p

        p_end = lax.fori_loop(0, (BPB + NBUF - 1) // NBUF, ring_body, lo)
        n = p_end - lo

        def obody(t, carry):
            @pl.when(t < n)
            def _():
                pltpu.async_copy(
                    slab_v.at[t >> 1, pl.ds((t & 1) * D, D)],
                    out_hbm.at[pl.ds((lo + t) * D, D)],
                    osem,
                )
            return carry

        lax.fori_loop(0, SLAB, obody, 0)

        def dbody(t, carry):
            @pl.when(t < n)
            def _():
                pltpu.make_async_copy(
                    out_hbm.at[pl.ds(0, D)],
                    slab_v.at[0, pl.ds(0, D)],
                    osem,
                ).wait()
            return carry

        lax.fori_loop(0, SLAB, dbody, 0)

    return k(tableT, tailT, sidx, bounds)


def _sc_unsort(spair, rank):
    mesh = plsc.VectorSubcoreMesh(core_axis_name="c", subcore_axis_name="s")

    @functools.partial(
        pl.kernel,
        out_type=jax.ShapeDtypeStruct((B, D), jnp.float32),
        mesh=mesh,
        scratch_types=[
            pltpu.VMEM((BPW,), jnp.int32),
            pltpu.VMEM((NCH, CH), jnp.int32),
            pltpu.VMEM((BPW,), jnp.int32),
            pltpu.VMEM((2, CH, 2 * D), jnp.float32),
            pltpu.VMEM((BPW, D), jnp.float32),
            [pltpu.SemaphoreType.DMA] * 2,
        ],
        compiler_params=pltpu.CompilerParams(needs_layout_passes=False),
    )
    def k(tab_hbm, idx_hbm, out_hbm, idx_v, q_v, r_v, pair_v, rows_v, sems):
        wid = lax.axis_index("s") * NC + lax.axis_index("c")
        base = wid * BPW
        pltpu.sync_copy(idx_hbm.at[pl.ds(base, BPW)], idx_v)

        def qb(g, carry):
            v = idx_v[pl.ds(g * 16, 16)]
            r_v[pl.ds(g * 16, 16)] = v & 1
            return carry

        lax.fori_loop(0, BPW // 16, qb, 0)
        for ch in range(NCH):
            def qb2(g, carry, ch=ch):
                v = idx_v[pl.ds(ch * CH + g * 16, 16)]
                q_v[ch, pl.ds(g * 16, 16)] = v >> 1
                return carry
            lax.fori_loop(0, CH // 16, qb2, 0)

        def issue(ch):
            pltpu.async_copy(
                tab_hbm.at[q_v.at[ch]], pair_v.at[ch % 2], sems[ch % 2]
            )

        issue(0)
        for ch in range(NCH):
            buf = ch % 2
            pltpu.make_async_copy(
                tab_hbm.at[q_v.at[ch]], pair_v.at[buf], sems[buf]
            ).wait()
            if ch + 1 < NCH:
                issue(ch + 1)

            def ebody(t, carry, ch=ch, buf=buf):
                sel = plsc.load_gather(
                    r_v, [jnp.full((16,), ch * CH, jnp.int32) + t]
                )
                pick_lo = sel == 0
                for kk in range(D // 16):
                    lo = pair_v[buf, t, pl.ds(kk * 16, 16)]
                    hi = pair_v[buf, t, pl.ds(D + kk * 16, 16)]
                    rows_v[ch * CH + t, pl.ds(kk * 16, 16)] = jnp.where(
                        pick_lo, lo, hi
                    )
                return carry

            lax.fori_loop(0, CH, ebody, 0)

        pltpu.sync_copy(rows_v, out_hbm.at[pl.ds(base, BPW)])

    return k(spair, rank)


def _mlp_body(x_ref, w1_ref, b1_ref, w2_ref, b2_ref, oe_ref, os_ref):
    x = x_ref[...]
    h = jnp.dot(x, w1_ref[...], preferred_element_type=jnp.float32)
    h = jnp.maximum(h + b1_ref[...], 0.0)
    y = (
        jnp.dot(h, w2_ref[...], preferred_element_type=jnp.float32)
        + b2_ref[...]
    )
    oe_ref[...] = x.T
    os_ref[...] = y.T


def _tc_mlp(embs, W1, b1, W2, b2):
    bm = 2048
    return pl.pallas_call(
        _mlp_body,
        grid=(B // bm,),
        in_specs=[
            pl.BlockSpec((bm, D), lambda i: (i, 0)),
            pl.BlockSpec((D, H), lambda i: (0, 0)),
            pl.BlockSpec((1, H), lambda i: (0, 0)),
            pl.BlockSpec((H, D), lambda i: (0, 0)),
            pl.BlockSpec((1, D), lambda i: (0, 0)),
        ],
        out_specs=[
            pl.BlockSpec((D, bm), lambda i: (0, i)),
            pl.BlockSpec((D, bm), lambda i: (0, i)),
        ],
        out_shape=[
            jax.ShapeDtypeStruct((D, B), jnp.float32),
            jax.ShapeDtypeStruct((D, B), jnp.float32),
        ],
    )(embs, W1, b1.reshape(1, H), W2, b2.reshape(1, D))


def kernel(states, table, W1, b1, W2, b2):
    idx1 = states.reshape(B).astype(jnp.int32)
    iota = lax.iota(jnp.int32, B)
    sidx, perm = lax.sort([idx1, iota], num_keys=1)
    rank = jnp.zeros((B,), jnp.int32).at[perm].set(iota)
    starts = (lax.iota(jnp.int32, 48) * (BPB * 128)).astype(jnp.int32)
    bounds = jnp.searchsorted(sidx, starts).astype(jnp.int32)
    tableT = table.T
    tailT = jnp.pad(tableT[:, V - 64:], ((0, 0), (0, 64)))
    sidx_p = jnp.pad(sidx, (0, 64))
    sorted_flat = _sc_sweep(tableT, tailT, sidx_p, bounds)
    spair = sorted_flat.reshape(B * D // 128, 128)
    embs = _sc_unsort(spair, rank)
    embsT, sfsT = _tc_mlp(embs, W1, b1, W2, b2)
    return (embsT.T, sfsT.T)
